# Initial kernel scaffold; baseline (speedup 1.0000x reference)
import jax
import jax.numpy as jnp
from jax.experimental import pallas as pl


def kernel(pst_idx, color_sign, sob_sign, wtm, T_fs, T_va, T_ha, T_ra, W_tempo):
    raise NotImplementedError("write your pallas kernel here")


# speedup vs baseline: 17.2571x; 17.2571x over previous
import functools

import jax
import jax.numpy as jnp
from jax.experimental import pallas as pl
from jax.experimental.pallas import tpu as pltpu

B = 4096
L = 32
V = 185
PAD = 184
S1 = 256
S2 = 64
K = 256
BB = 512


def _tc_body(idx_ref, color_ref, sob_ref, wtm_ref, t1_ref, t2_ref, t3_ref,
             t4_ref, wt_ref, o1_ref, o2_ref, o3_ref, o4_ref):
    idx = idx_ref[...]
    color = color_ref[...]
    sob = sob_ref[...]
    iota = jax.lax.broadcasted_iota(jnp.int32, (BB, K), 1)

    h1 = jnp.zeros((BB, K), jnp.float32)
    h2 = jnp.zeros((BB, K), jnp.float32)
    h3 = jnp.zeros((BB, K), jnp.float32)
    h4 = jnp.zeros((BB, K), jnp.float32)
    for l in range(L):
        v = idx[:, l][:, None]
        c = color[:, l][:, None]
        s = sob[:, l][:, None]
        m = iota == v
        h1 = h1 + m.astype(jnp.float32)
        h2 = h2 + jnp.where(m, c, 0.0)
        h3 = h3 + jnp.where(m, s, 0.0)
        h4 = h4 + jnp.where(m, c * s, 0.0)

    o1_ref[...] = jnp.dot(h1, t1_ref[...], preferred_element_type=jnp.float32)
    o2_ref[...] = (jnp.dot(h2, t2_ref[...], preferred_element_type=jnp.float32)
                   + wtm_ref[...] * wt_ref[...])
    o3_ref[...] = jnp.dot(h3, t3_ref[...], preferred_element_type=jnp.float32)
    o4_ref[...] = jnp.dot(h4, t4_ref[...], preferred_element_type=jnp.float32)


@jax.jit
def kernel(pst_idx, color_sign, sob_sign, wtm, T_fs, T_va, T_ha, T_ra,
           W_tempo):
    t1 = jnp.zeros((K, S1), jnp.float32).at[:V].set(T_fs)
    t2 = jnp.zeros((K, S1), jnp.float32).at[:V].set(T_va)
    t3 = jnp.zeros((K, S2), jnp.float32).at[:V].set(T_ha)
    t4 = jnp.zeros((K, S2), jnp.float32).at[:V].set(T_ra)
    wt = W_tempo.reshape(1, S1)

    grid = (B // BB,)
    bspec = lambda bs, d: pl.BlockSpec((BB, d), lambda i: (i, 0))
    tspec = lambda d: pl.BlockSpec((K, d), lambda i: (0, 0))
    out = pl.pallas_call(
        _tc_body,
        grid=grid,
        in_specs=[
            pl.BlockSpec((BB, L), lambda i: (i, 0)),
            pl.BlockSpec((BB, L), lambda i: (i, 0)),
            pl.BlockSpec((BB, L), lambda i: (i, 0)),
            pl.BlockSpec((BB, 1), lambda i: (i, 0)),
            tspec(S1), tspec(S1), tspec(S2), tspec(S2),
            pl.BlockSpec((1, S1), lambda i: (0, 0)),
        ],
        out_specs=[
            pl.BlockSpec((BB, S1), lambda i: (i, 0)),
            pl.BlockSpec((BB, S1), lambda i: (i, 0)),
            pl.BlockSpec((BB, S2), lambda i: (i, 0)),
            pl.BlockSpec((BB, S2), lambda i: (i, 0)),
        ],
        out_shape=[
            jax.ShapeDtypeStruct((B, S1), jnp.float32),
            jax.ShapeDtypeStruct((B, S1), jnp.float32),
            jax.ShapeDtypeStruct((B, S2), jnp.float32),
            jax.ShapeDtypeStruct((B, S2), jnp.float32),
        ],
    )(pst_idx, color_sign, sob_sign, wtm, t1, t2, t3, t4, wt)
    return tuple(out)


# speedup vs baseline: 35.2549x; 2.0429x over previous
import functools

import jax
import jax.numpy as jnp
from jax import lax
from jax.experimental import pallas as pl
from jax.experimental.pallas import tpu as pltpu
from jax.experimental.pallas import tpu_sc as plsc

B = 4096
L = 32
V = 185
PAD = 184
S1 = 256
S2 = 64
K = 256
BB = 512
NW = 32
SPT = B // NW
RND = 64


def _sc_hist_body(idx_hbm, col_hbm, sob_hbm, h_hbm, idx_v, col_v, sob_v, h_v):
    c = lax.axis_index("c")
    s = lax.axis_index("s")
    wid = s * 2 + c
    base = wid * SPT
    pltpu.sync_copy(idx_hbm.at[wid], idx_v)
    pltpu.sync_copy(col_hbm.at[wid], col_v)
    pltpu.sync_copy(sob_hbm.at[wid], sob_v)
    iota16 = lax.iota(jnp.int32, 16)
    ones16 = jnp.ones((16,), jnp.float32)
    z16 = jnp.zeros((16,), jnp.float32)
    k16 = [jnp.full((16,), k, jnp.int32) for k in range(4)]
    for r in range(2):
        def zero_body(b, carry):
            for k in range(4):
                for j in range(K // 16):
                    h_v[k, b, pl.ds(j * 16, 16)] = z16
            return carry
        lax.fori_loop(0, RND, zero_body, 0)

        def scat_body(l, carry):
            for chunk in range(RND // 16):
                off = r * RND + chunk * 16
                vi = idx_v[l, pl.ds(off, 16)]
                cv = col_v[l, pl.ds(off, 16)]
                sv = sob_v[l, pl.ds(off, 16)]
                b16 = chunk * 16 + iota16
                plsc.addupdate_scatter(h_v, [k16[0], b16, vi], ones16)
                plsc.addupdate_scatter(h_v, [k16[1], b16, vi], cv)
                plsc.addupdate_scatter(h_v, [k16[2], b16, vi], sv)
                plsc.addupdate_scatter(h_v, [k16[3], b16, vi], cv * sv)
            return carry
        lax.fori_loop(0, L, scat_body, 0)

        for k in range(4):
            pltpu.sync_copy(h_v.at[k], h_hbm.at[k, pl.ds(base + r * RND, RND)])


def _tc_mm_body(h_ref, wtm_ref, t1_ref, t2_ref, t3_ref, t4_ref, wt_ref,
                o1_ref, o2_ref, o3_ref, o4_ref):
    h = h_ref[...]
    o1_ref[...] = jnp.dot(h[0], t1_ref[...], preferred_element_type=jnp.float32)
    o2_ref[...] = (jnp.dot(h[1], t2_ref[...], preferred_element_type=jnp.float32)
                   + wtm_ref[...] * wt_ref[...])
    o3_ref[...] = jnp.dot(h[2], t3_ref[...], preferred_element_type=jnp.float32)
    o4_ref[...] = jnp.dot(h[3], t4_ref[...], preferred_element_type=jnp.float32)


@jax.jit
def kernel(pst_idx, color_sign, sob_sign, wtm, T_fs, T_va, T_ha, T_ra,
           W_tempo):
    idx3 = pst_idx.reshape(NW, SPT, L).transpose(0, 2, 1)
    col3 = color_sign.reshape(NW, SPT, L).transpose(0, 2, 1)
    sob3 = sob_sign.reshape(NW, SPT, L).transpose(0, 2, 1)

    mesh = plsc.VectorSubcoreMesh(core_axis_name="c", subcore_axis_name="s")
    hist = pl.kernel(
        _sc_hist_body,
        out_type=jax.ShapeDtypeStruct((4, B, K), jnp.float32),
        mesh=mesh,
        compiler_params=pltpu.CompilerParams(needs_layout_passes=False),
        scratch_types=[
            pltpu.VMEM((L, SPT), jnp.int32),
            pltpu.VMEM((L, SPT), jnp.float32),
            pltpu.VMEM((L, SPT), jnp.float32),
            pltpu.VMEM((4, RND, K), jnp.float32),
        ],
    )(idx3, col3, sob3)

    t1 = jnp.zeros((K, S1), jnp.float32).at[:V].set(T_fs)
    t2 = jnp.zeros((K, S1), jnp.float32).at[:V].set(T_va)
    t3 = jnp.zeros((K, S2), jnp.float32).at[:V].set(T_ha)
    t4 = jnp.zeros((K, S2), jnp.float32).at[:V].set(T_ra)
    wt = W_tempo.reshape(1, S1)

    tspec = lambda d: pl.BlockSpec((K, d), lambda i: (0, 0))
    out = pl.pallas_call(
        _tc_mm_body,
        grid=(B // BB,),
        in_specs=[
            pl.BlockSpec((4, BB, K), lambda i: (0, i, 0)),
            pl.BlockSpec((BB, 1), lambda i: (i, 0)),
            tspec(S1), tspec(S1), tspec(S2), tspec(S2),
            pl.BlockSpec((1, S1), lambda i: (0, 0)),
        ],
        out_specs=[
            pl.BlockSpec((BB, S1), lambda i: (i, 0)),
            pl.BlockSpec((BB, S1), lambda i: (i, 0)),
            pl.BlockSpec((BB, S2), lambda i: (i, 0)),
            pl.BlockSpec((BB, S2), lambda i: (i, 0)),
        ],
        out_shape=[
            jax.ShapeDtypeStruct((B, S1), jnp.float32),
            jax.ShapeDtypeStruct((B, S1), jnp.float32),
            jax.ShapeDtypeStruct((B, S2), jnp.float32),
            jax.ShapeDtypeStruct((B, S2), jnp.float32),
        ],
    )(hist, wtm, t1, t2, t3, t4, wt)
    return tuple(out)
